# baseline (speedup 1.0000x reference)
import functools

import jax
import jax.numpy as jnp
from jax import lax
from jax.experimental import pallas as pl
from jax.experimental.pallas import tpu as pltpu
from jax.experimental.pallas import tpu_sc as plsc

N = 10000
E = 320000
D = 128
H = 128
HH = H // 2

_NC = 2
_NS = 16
_EPT = E // _NS
_B = 125
_NCH = _EPT // _B
_NP = 10240
_RP = _NP // _NS
_NBUF = 4
_PF = _NBUF // 2

_EPS = 1e-5


def _bn(h, g, b):
    mu = jnp.mean(h, axis=0, keepdims=True)
    var = jnp.mean((h - mu) ** 2, axis=0, keepdims=True)
    return (h - mu) / jnp.sqrt(var + _EPS) * g + b


def _matmul_t(a, w):
    return lax.dot_general(a, w, (((1,), (1,)), ((), ())),
                           preferred_element_type=jnp.float32)


def _mlp_body(x_ref, wi_ref, bi_ref, gi_ref, bei_ref, wh_ref, bh_ref,
              gh_ref, beh_ref, feat_ref, feath_ref):
    x = x_ref[...]
    h = _bn(_matmul_t(x, wi_ref[...]) + bi_ref[...], gi_ref[...], bei_ref[...])
    h = jnp.maximum(h, 0.0)
    h = _bn(_matmul_t(h, wh_ref[...]) + bh_ref[...], gh_ref[...], beh_ref[...])
    h = jnp.maximum(h, 0.0)
    feat_ref[...] = h
    feath_ref[0] = h[:, :HH]
    feath_ref[1] = h[:, HH:]


_mlp_call = pl.pallas_call(
    _mlp_body,
    out_shape=(jax.ShapeDtypeStruct((N, H), jnp.float32),
               jax.ShapeDtypeStruct((_NC, N, HH), jnp.float32)),
)


def _make_sage_tc(emit_halves):
    def body(pagg_ref, pcnt_ref, h_ref, wl_ref, bl_ref, wr_ref, g_ref,
             be_ref, out_ref, *outh):
        cnt = (pcnt_ref[0] + pcnt_ref[1])[:N, 0:1]
        agg = jnp.concatenate([pagg_ref[0, :N], pagg_ref[1, :N]], axis=1)
        agg = agg / jnp.maximum(cnt, 1.0)
        o = _matmul_t(agg, wl_ref[...]) + bl_ref[...]
        o = o + _matmul_t(h_ref[...], wr_ref[...])
        o = _bn(o, g_ref[...], be_ref[...])
        out_ref[...] = o
        if emit_halves:
            outh[0][0] = o[:, :HH]
            outh[0][1] = o[:, HH:]

    if emit_halves:
        out_shape = (jax.ShapeDtypeStruct((N, H), jnp.float32),
                     jax.ShapeDtypeStruct((_NC, N, HH), jnp.float32))
    else:
        out_shape = jax.ShapeDtypeStruct((N, H), jnp.float32)
    return pl.pallas_call(body, out_shape=out_shape)


_sage_tc_h = _make_sage_tc(True)
_sage_tc = _make_sage_tc(False)


_mesh = plsc.VectorSubcoreMesh(core_axis_name="c", subcore_axis_name="s")


def _make_sc_agg(with_counts):
    if with_counts:
        out_type = (jax.ShapeDtypeStruct((_NC, _NP, HH), jnp.float32),
                    jax.ShapeDtypeStruct((_NC, _NP, 16), jnp.float32))
    else:
        out_type = jax.ShapeDtypeStruct((_NC, _NP, HH), jnp.float32)

    scratch_types = [
        pltpu.VMEM((_NCH, _B), jnp.int32),
        pltpu.VMEM((_NCH, _B), jnp.int32),
        pltpu.VMEM((_NBUF, _B, HH), jnp.float32),
        pltpu.VMEM((_B, 16), jnp.float32),
        pltpu.VMEM_SHARED((_NP, HH), jnp.float32),
        pltpu.VMEM_SHARED((_NP, 16), jnp.float32),
        pltpu.SemaphoreType.DMA((_NBUF,)),
        pltpu.SemaphoreType.DMA((_NBUF,)),
    ]

    @functools.partial(
        pl.kernel, out_type=out_type, scratch_types=scratch_types,
        mesh=_mesh,
        compiler_params=pltpu.CompilerParams(use_tc_tiling_on_sc=False))
    def sc_agg(src_hbm, dst_hbm, feath_hbm, zf_hbm, zc_hbm, ones_hbm,
               *rest):
        if with_counts:
            (pagg_hbm, pcnt_hbm, src_v, dst_v, rows_v, ones_v, agg_s,
             cnt_s, gsem, ssem) = rest
        else:
            (pagg_hbm, src_v, dst_v, rows_v, ones_v, agg_s, cnt_s,
             gsem, ssem) = rest
        c = lax.axis_index("c")
        s = lax.axis_index("s")

        pltpu.sync_copy(src_hbm.at[s], src_v)
        pltpu.sync_copy(dst_hbm.at[s], dst_v)
        pltpu.sync_copy(zf_hbm, agg_s.at[pl.ds(s * _RP, _RP)])
        if with_counts:
            pltpu.sync_copy(ones_hbm, ones_v)
            pltpu.sync_copy(zc_hbm, cnt_s.at[pl.ds(s * _RP, _RP)])
        plsc.subcore_barrier()

        def fire_gather(j, b):
            pltpu.async_copy(feath_hbm.at[c].at[src_v.at[j]], rows_v.at[b],
                             gsem.at[b])

        def wait_gather(j, b):
            pltpu.make_async_copy(feath_hbm.at[c].at[src_v.at[j]],
                                  rows_v.at[b], gsem.at[b]).wait()

        def wait_scatter(b):
            pltpu.make_async_copy(rows_v.at[b], agg_s.at[dst_v.at[0]],
                                  ssem.at[b]).wait()

        for b in range(_PF):
            fire_gather(b, b)

        def body(g, carry):
            base = g * _NBUF
            for b in range(_NBUF):
                j = base + b
                wait_gather(j, b)
                pltpu.async_copy(rows_v.at[b], agg_s.at[dst_v.at[j]],
                                 ssem.at[b], add=True)
                if with_counts:
                    @pl.when(jnp.where(c == 0, j < _NCH // 2,
                                       j >= _NCH // 2))
                    def _():
                        pltpu.sync_copy(ones_v, cnt_s.at[dst_v.at[j]],
                                        add=True)

                b2 = (b + _PF) % _NBUF
                @pl.when(j >= _PF)
                def _():
                    wait_scatter(b2)

                @pl.when(j + _PF < _NCH)
                def _():
                    fire_gather(j + _PF, b2)
            return carry

        lax.fori_loop(0, _NCH // _NBUF, body, 0)

        for k in range(_NCH - _PF, _NCH):
            wait_scatter(k % _NBUF)
        plsc.subcore_barrier()

        rs = pl.ds(s * _RP, _RP)
        pltpu.sync_copy(agg_s.at[rs], pagg_hbm.at[c, rs])
        if with_counts:
            pltpu.sync_copy(cnt_s.at[rs], pcnt_hbm.at[c, rs])

    return sc_agg


_sc_agg_counts = _make_sc_agg(True)
_sc_agg_plain = _make_sc_agg(False)


def kernel(x, edge_index, W_in, b_in, g_in, be_in, W_hid, b_hid, g_hid,
           be_hid, Wl1, bl1, Wr1, g1, be1, Wl2, bl2, Wr2, g2, be2):
    row = lambda v: v.reshape(1, -1)
    feat, feat_h = _mlp_call(x, W_in, row(b_in), row(g_in), row(be_in),
                             W_hid, row(b_hid), row(g_hid), row(be_hid))

    src_r = edge_index[0].reshape(_NS, _NCH, _B)
    dst_r = edge_index[1].reshape(_NS, _NCH, _B)
    zf = jnp.zeros((_RP, HH), jnp.float32)
    zc = jnp.zeros((_RP, 16), jnp.float32)
    ones = jnp.ones((_B, 16), jnp.float32)

    pagg1, pcnt = _sc_agg_counts(src_r, dst_r, feat_h, zf, zc, ones)
    out1, out1_h = _sage_tc_h(pagg1, pcnt, feat, Wl1, row(bl1), Wr1,
                              row(g1), row(be1))
    pagg2 = _sc_agg_plain(src_r, dst_r, out1_h, zf, zc, ones)
    out2 = _sage_tc(pagg2, pcnt, out1, Wl2, row(bl2), Wr2, row(g2),
                    row(be2))
    return feat, out2


# speedup vs baseline: 1.1335x; 1.1335x over previous
import functools

import jax
import jax.numpy as jnp
from jax import lax
from jax.experimental import pallas as pl
from jax.experimental.pallas import tpu as pltpu
from jax.experimental.pallas import tpu_sc as plsc

N = 10000
E = 320000
D = 128
H = 128
HH = H // 2

_NC = 2
_NS = 16
_EPT = E // _NS
_B = 125
_NCH = _EPT // _B
_NP = 10240
_RP = _NP // _NS
_NBUF = 5
_PF = 3
_SLACK = _NBUF - _PF

_EPS = 1e-5


def _bn(h, g, b):
    mu = jnp.mean(h, axis=0, keepdims=True)
    var = jnp.mean((h - mu) ** 2, axis=0, keepdims=True)
    return (h - mu) / jnp.sqrt(var + _EPS) * g + b


def _matmul_t(a, w):
    return lax.dot_general(a, w, (((1,), (1,)), ((), ())),
                           preferred_element_type=jnp.float32)


def _mlp_body(x_ref, wi_ref, bi_ref, gi_ref, bei_ref, wh_ref, bh_ref,
              gh_ref, beh_ref, feat_ref, feath_ref):
    x = x_ref[...]
    h = _bn(_matmul_t(x, wi_ref[...]) + bi_ref[...], gi_ref[...], bei_ref[...])
    h = jnp.maximum(h, 0.0)
    h = _bn(_matmul_t(h, wh_ref[...]) + bh_ref[...], gh_ref[...], beh_ref[...])
    h = jnp.maximum(h, 0.0)
    feat_ref[...] = h
    feath_ref[0] = h[:, :HH]
    feath_ref[1] = h[:, HH:]


_mlp_call = pl.pallas_call(
    _mlp_body,
    out_shape=(jax.ShapeDtypeStruct((N, H), jnp.float32),
               jax.ShapeDtypeStruct((_NC, N, HH), jnp.float32)),
)


def _make_sage_tc(emit_halves):
    def body(pagg_ref, pcnt_ref, h_ref, wl_ref, bl_ref, wr_ref, g_ref,
             be_ref, out_ref, *outh):
        cnt = (pcnt_ref[0] + pcnt_ref[1])[:N, 0:1]
        agg = jnp.concatenate([pagg_ref[0, :N], pagg_ref[1, :N]], axis=1)
        agg = agg / jnp.maximum(cnt, 1.0)
        o = _matmul_t(agg, wl_ref[...]) + bl_ref[...]
        o = o + _matmul_t(h_ref[...], wr_ref[...])
        o = _bn(o, g_ref[...], be_ref[...])
        out_ref[...] = o
        if emit_halves:
            outh[0][0] = o[:, :HH]
            outh[0][1] = o[:, HH:]

    if emit_halves:
        out_shape = (jax.ShapeDtypeStruct((N, H), jnp.float32),
                     jax.ShapeDtypeStruct((_NC, N, HH), jnp.float32))
    else:
        out_shape = jax.ShapeDtypeStruct((N, H), jnp.float32)
    return pl.pallas_call(body, out_shape=out_shape)


_sage_tc_h = _make_sage_tc(True)
_sage_tc = _make_sage_tc(False)


_mesh = plsc.VectorSubcoreMesh(core_axis_name="c", subcore_axis_name="s")


_agg_scratch = [
    pltpu.VMEM((_NCH, _B), jnp.int32),
    pltpu.VMEM((_NCH, _B), jnp.int32),
    pltpu.VMEM((_NBUF, _B, HH), jnp.float32),
    pltpu.VMEM_SHARED((_NP, HH), jnp.float32),
    pltpu.SemaphoreType.DMA((_NBUF,)),
    pltpu.SemaphoreType.DMA((_NBUF,)),
]


@functools.partial(
    pl.kernel,
    out_type=jax.ShapeDtypeStruct((_NC, _NP, HH), jnp.float32),
    scratch_types=_agg_scratch, mesh=_mesh,
    compiler_params=pltpu.CompilerParams(use_tc_tiling_on_sc=False))
def _sc_agg(src_hbm, dst_hbm, feath_hbm, zf_hbm, pagg_hbm, src_v, dst_v,
            rows_v, agg_s, gsem, ssem):
    c = lax.axis_index("c")
    s = lax.axis_index("s")

    pltpu.sync_copy(src_hbm.at[s], src_v)
    pltpu.sync_copy(dst_hbm.at[s], dst_v)
    pltpu.sync_copy(zf_hbm, agg_s.at[pl.ds(s * _RP, _RP)])
    plsc.subcore_barrier()

    def fire_gather(j, b):
        pltpu.async_copy(feath_hbm.at[c].at[src_v.at[j]], rows_v.at[b],
                         gsem.at[b])

    def wait_gather(j, b):
        pltpu.make_async_copy(feath_hbm.at[c].at[src_v.at[j]],
                              rows_v.at[b], gsem.at[b]).wait()

    def wait_scatter(b):
        pltpu.make_async_copy(rows_v.at[b], agg_s.at[dst_v.at[0]],
                              ssem.at[b]).wait()

    for b in range(_PF):
        fire_gather(b, b)

    def body(g, carry):
        base = g * _NBUF
        for b in range(_NBUF):
            j = base + b
            wait_gather(j, b)
            pltpu.async_copy(rows_v.at[b], agg_s.at[dst_v.at[j]],
                             ssem.at[b], add=True)
            b2 = (b + _PF) % _NBUF
            @pl.when(j >= _SLACK)
            def _():
                wait_scatter(b2)

            @pl.when(j + _PF < _NCH)
            def _():
                fire_gather(j + _PF, b2)
        return carry

    lax.fori_loop(0, _NCH // _NBUF, body, 0)

    for k in range(_NCH - _SLACK, _NCH):
        wait_scatter(k % _NBUF)
    plsc.subcore_barrier()

    rs = pl.ds(s * _RP, _RP)
    pltpu.sync_copy(agg_s.at[rs], pagg_hbm.at[c, rs])


_CNBUF = 4

_cnt_scratch = [
    pltpu.VMEM((_NCH, _B), jnp.int32),
    pltpu.VMEM((_B, 16), jnp.float32),
    pltpu.VMEM_SHARED((_NP, 16), jnp.float32),
    pltpu.SemaphoreType.DMA((_CNBUF,)),
]


@functools.partial(
    pl.kernel,
    out_type=jax.ShapeDtypeStruct((_NC, _NP, 16), jnp.float32),
    scratch_types=_cnt_scratch, mesh=_mesh,
    compiler_params=pltpu.CompilerParams(use_tc_tiling_on_sc=False))
def _sc_cnt(dst_hbm, zc_hbm, ones_hbm, pcnt_hbm, dst_v, ones_v, cnt_s,
            csem):
    c = lax.axis_index("c")
    s = lax.axis_index("s")

    pltpu.sync_copy(dst_hbm.at[s], dst_v)
    pltpu.sync_copy(ones_hbm, ones_v)
    pltpu.sync_copy(zc_hbm, cnt_s.at[pl.ds(s * _RP, _RP)])
    plsc.subcore_barrier()

    half = _NCH // 2
    lo = c * half

    def wait_cnt(j):
        pltpu.make_async_copy(ones_v, cnt_s.at[dst_v.at[0]],
                              csem.at[j % _CNBUF]).wait()

    def body(g, carry):
        for b in range(_CNBUF):
            j = g * _CNBUF + b

            @pl.when(j >= _CNBUF)
            def _():
                wait_cnt(j)
            pltpu.async_copy(ones_v, cnt_s.at[dst_v.at[lo + j]],
                             csem.at[b], add=True)
        return carry

    lax.fori_loop(0, half // _CNBUF, body, 0)
    for k in range(half - _CNBUF, half):
        wait_cnt(k)
    plsc.subcore_barrier()

    rs = pl.ds(s * _RP, _RP)
    pltpu.sync_copy(cnt_s.at[rs], pcnt_hbm.at[c, rs])


def kernel(x, edge_index, W_in, b_in, g_in, be_in, W_hid, b_hid, g_hid,
           be_hid, Wl1, bl1, Wr1, g1, be1, Wl2, bl2, Wr2, g2, be2):
    row = lambda v: v.reshape(1, -1)
    feat, feat_h = _mlp_call(x, W_in, row(b_in), row(g_in), row(be_in),
                             W_hid, row(b_hid), row(g_hid), row(be_hid))

    src_r = edge_index[0].reshape(_NS, _NCH, _B)
    dst_r = edge_index[1].reshape(_NS, _NCH, _B)
    zf = jnp.zeros((_RP, HH), jnp.float32)
    zc = jnp.zeros((_RP, 16), jnp.float32)
    ones = jnp.ones((_B, 16), jnp.float32)

    pcnt = _sc_cnt(dst_r, zc, ones)
    pagg1 = _sc_agg(src_r, dst_r, feat_h, zf)
    out1, out1_h = _sage_tc_h(pagg1, pcnt, feat, Wl1, row(bl1), Wr1,
                              row(g1), row(be1))
    pagg2 = _sc_agg(src_r, dst_r, out1_h, zf)
    out2 = _sage_tc(pagg2, pcnt, out1, Wl2, row(bl2), Wr2, row(g2),
                    row(be2))
    return feat, out2


# speedup vs baseline: 1.3487x; 1.1899x over previous
import functools

import jax
import jax.numpy as jnp
from jax import lax
from jax.experimental import pallas as pl
from jax.experimental.pallas import tpu as pltpu
from jax.experimental.pallas import tpu_sc as plsc

N = 10000
E = 320000
D = 128
H = 128
HH = H // 2

_NC = 2
_NS = 16
_EPT = E // _NS
_B = 125
_NCH = _EPT // _B
_NP = 10240
_RP = _NP // _NS
_NBUF = 5
_PF = 3
_SLACK = _NBUF - _PF

_EPS = 1e-5


def _bn(h, g, b):
    mu = jnp.mean(h, axis=0, keepdims=True)
    var = jnp.mean((h - mu) ** 2, axis=0, keepdims=True)
    return (h - mu) / jnp.sqrt(var + _EPS) * g + b


def _matmul_t(a, w):
    return lax.dot_general(a, w, (((1,), (1,)), ((), ())),
                           preferred_element_type=jnp.float32)


def _pack_halves(h, feath_ref):
    h3 = h.reshape(N // 2, 2, H)
    even, odd = h3[:, 0, :], h3[:, 1, :]
    feath_ref[0] = jnp.concatenate([even[:, :HH], odd[:, :HH]], axis=1)
    feath_ref[1] = jnp.concatenate([even[:, HH:], odd[:, HH:]], axis=1)


def _mlp_body(x_ref, wi_ref, bi_ref, gi_ref, bei_ref, wh_ref, bh_ref,
              gh_ref, beh_ref, feat_ref, feath_ref):
    x = x_ref[...]
    h = _bn(_matmul_t(x, wi_ref[...]) + bi_ref[...], gi_ref[...], bei_ref[...])
    h = jnp.maximum(h, 0.0)
    h = _bn(_matmul_t(h, wh_ref[...]) + bh_ref[...], gh_ref[...], beh_ref[...])
    h = jnp.maximum(h, 0.0)
    feat_ref[...] = h
    _pack_halves(h, feath_ref)


_mlp_call = pl.pallas_call(
    _mlp_body,
    out_shape=(jax.ShapeDtypeStruct((N, H), jnp.float32),
               jax.ShapeDtypeStruct((_NC, N // 2, H), jnp.float32)),
)


def _make_sage_tc(emit_halves):
    def body(pagg_ref, pcnt_ref, h_ref, wl_ref, bl_ref, wr_ref, g_ref,
             be_ref, out_ref, *outh):
        cnt = (pcnt_ref[0] + pcnt_ref[1])[:N, 0:1]
        agg = pagg_ref[:N] / jnp.maximum(cnt, 1.0)
        o = _matmul_t(agg, wl_ref[...]) + bl_ref[...]
        o = o + _matmul_t(h_ref[...], wr_ref[...])
        o = _bn(o, g_ref[...], be_ref[...])
        out_ref[...] = o
        if emit_halves:
            _pack_halves(o, outh[0])

    if emit_halves:
        out_shape = (jax.ShapeDtypeStruct((N, H), jnp.float32),
                     jax.ShapeDtypeStruct((_NC, N // 2, H), jnp.float32))
    else:
        out_shape = jax.ShapeDtypeStruct((N, H), jnp.float32)
    return pl.pallas_call(body, out_shape=out_shape)


_sage_tc_h = _make_sage_tc(True)
_sage_tc = _make_sage_tc(False)


_mesh = plsc.VectorSubcoreMesh(core_axis_name="c", subcore_axis_name="s")


_agg_scratch = [
    pltpu.VMEM((_NCH, _B), jnp.int32),
    pltpu.VMEM((_NCH, _B), jnp.int32),
    pltpu.VMEM((_NBUF, _B, HH), jnp.float32),
    pltpu.VMEM_SHARED((_NP, HH), jnp.float32),
    pltpu.SemaphoreType.DMA((_NBUF,)),
    pltpu.SemaphoreType.DMA((_NBUF,)),
]


@functools.partial(
    pl.kernel,
    out_type=jax.ShapeDtypeStruct((_NP, H), jnp.float32),
    scratch_types=_agg_scratch, mesh=_mesh,
    compiler_params=pltpu.CompilerParams(use_tc_tiling_on_sc=False))
def _sc_agg(er_hbm, feath_hbm, zf_hbm, pagg_hbm, src_v, dst_v,
            rows_v, agg_s, gsem, ssem):
    c = lax.axis_index("c")
    s = lax.axis_index("s")

    feat_half = feath_hbm.at[c]

    pltpu.sync_copy(er_hbm.at[0, s], src_v)
    pltpu.sync_copy(er_hbm.at[1, s], dst_v)
    pltpu.sync_copy(zf_hbm, agg_s.at[pl.ds(s * _RP, _RP)])
    plsc.subcore_barrier()

    def fire_gather(j, b):
        pltpu.async_copy(feat_half.at[src_v.at[j]], rows_v.at[b],
                         gsem.at[b])

    def wait_gather(j, b):
        pltpu.make_async_copy(feat_half.at[src_v.at[j]],
                              rows_v.at[b], gsem.at[b]).wait()

    def wait_scatter(b):
        pltpu.make_async_copy(rows_v.at[b], agg_s.at[dst_v.at[0]],
                              ssem.at[b]).wait()

    for b in range(_PF):
        fire_gather(b, b)

    def body(g, carry):
        base = g * _NBUF
        for b in range(_NBUF):
            j = base + b
            wait_gather(j, b)
            pltpu.async_copy(rows_v.at[b], agg_s.at[dst_v.at[j]],
                             ssem.at[b], add=True)
            b2 = (b + _PF) % _NBUF
            @pl.when(j >= _SLACK)
            def _():
                wait_scatter(b2)

            @pl.when(j + _PF < _NCH)
            def _():
                fire_gather(j + _PF, b2)
        return carry

    lax.fori_loop(0, _NCH // _NBUF, body, 0)

    for k in range(_NCH - _SLACK, _NCH):
        wait_scatter(k % _NBUF)
    plsc.subcore_barrier()

    rs = pl.ds(s * _RP, _RP)
    pltpu.sync_copy(agg_s.at[rs], pagg_hbm.at[rs, pl.ds(c * HH, HH)])


_CNBUF = 4

_cnt_scratch = [
    pltpu.VMEM((_NCH, _B), jnp.int32),
    pltpu.VMEM((_B, 16), jnp.float32),
    pltpu.VMEM_SHARED((_NP, 16), jnp.float32),
    pltpu.SemaphoreType.DMA((_CNBUF,)),
]


@functools.partial(
    pl.kernel,
    out_type=jax.ShapeDtypeStruct((_NC, _NP, 16), jnp.float32),
    scratch_types=_cnt_scratch, mesh=_mesh,
    compiler_params=pltpu.CompilerParams(use_tc_tiling_on_sc=False))
def _sc_cnt(er_hbm, zc_hbm, ones_hbm, pcnt_hbm, dst_v, ones_v, cnt_s,
            csem):
    c = lax.axis_index("c")
    s = lax.axis_index("s")

    pltpu.sync_copy(er_hbm.at[1, s], dst_v)
    pltpu.sync_copy(ones_hbm, ones_v)
    pltpu.sync_copy(zc_hbm, cnt_s.at[pl.ds(s * _RP, _RP)])
    plsc.subcore_barrier()

    half = _NCH // 2
    lo = c * half

    def wait_cnt(j):
        pltpu.make_async_copy(ones_v, cnt_s.at[dst_v.at[0]],
                              csem.at[j % _CNBUF]).wait()

    def body(g, carry):
        for b in range(_CNBUF):
            j = g * _CNBUF + b

            @pl.when(j >= _CNBUF)
            def _():
                wait_cnt(j)
            pltpu.async_copy(ones_v, cnt_s.at[dst_v.at[lo + j]],
                             csem.at[b], add=True)
        return carry

    lax.fori_loop(0, half // _CNBUF, body, 0)
    for k in range(half - _CNBUF, half):
        wait_cnt(k)
    plsc.subcore_barrier()

    rs = pl.ds(s * _RP, _RP)
    pltpu.sync_copy(cnt_s.at[rs], pcnt_hbm.at[c, rs])


def kernel(x, edge_index, W_in, b_in, g_in, be_in, W_hid, b_hid, g_hid,
           be_hid, Wl1, bl1, Wr1, g1, be1, Wl2, bl2, Wr2, g2, be2):
    row = lambda v: v.reshape(1, -1)
    feat, feat_h2 = _mlp_call(x, W_in, row(b_in), row(g_in), row(be_in),
                              W_hid, row(b_hid), row(g_hid), row(be_hid))

    er = edge_index.reshape(2, _NS, _NCH, _B)
    zf = jnp.zeros((_RP, HH), jnp.float32)
    zc = jnp.zeros((_RP, 16), jnp.float32)
    ones = jnp.ones((_B, 16), jnp.float32)

    pcnt = _sc_cnt(er, zc, ones)
    pagg1 = _sc_agg(er, feat_h2.reshape(_NC, N, HH), zf)
    out1, out1_h2 = _sage_tc_h(pagg1, pcnt, feat, Wl1, row(bl1), Wr1,
                               row(g1), row(be1))
    pagg2 = _sc_agg(er, out1_h2.reshape(_NC, N, HH), zf)
    out2 = _sage_tc(pagg2, pcnt, out1, Wl2, row(bl2), Wr2, row(g2),
                    row(be2))
    return feat, out2
